# Initial kernel scaffold; baseline (speedup 1.0000x reference)
import jax
import jax.numpy as jnp
from jax.experimental import pallas as pl


def kernel(x, edge_index, batch, W1, b1, W2, b2, W3, b3, Wf1, bf1, Wf2, bf2):
    raise NotImplementedError("write your pallas kernel here")


# speedup vs baseline: 7.4797x; 7.4797x over previous
import functools

import jax
import jax.numpy as jnp
from jax import lax
from jax.experimental import pallas as pl
from jax.experimental.pallas import tpu as pltpu
from jax.experimental.pallas import tpu_sc as plsc

N = 50000
G = 64
E = 800000

NC = 2
NS = 16
BSZ = 128
NBLK = 392
EPT = NBLK * BSZ
EP = NS * EPT
DEG_NBLK = EP // (NC * NS * BSZ)

NPAD = 51200
RBLK = 2048
NROWBLKS = NPAD // RBLK
ACCR = 50048
RPT = ACCR // NS
TAILR = NPAD - ACCR
TPT = TAILR // NS


@functools.cache
def _sc_mesh():
    return plsc.VectorSubcoreMesh(core_axis_name="c", subcore_axis_name="s",
                                  num_cores=NC, num_subcores=NS)


def _deg_body(dst_h, zero_h, d0_h, d1_h, dst_v, ones_v, acc, sem):
    cid = lax.axis_index("c")
    sid = lax.axis_index("s")
    for i in range(BSZ // 16):
        ones_v[pl.ds(i * 16, 16)] = jnp.ones((16,), jnp.float32)
    pltpu.sync_copy(zero_h.at[pl.ds(sid * RPT, RPT)],
                    acc.at[pl.ds(sid * RPT, RPT)])
    plsc.subcore_barrier()
    base = cid * (EP // NC) + sid * (DEG_NBLK * BSZ)

    @pl.loop(0, DEG_NBLK)
    def _acc(j):
        pltpu.async_copy(dst_h.at[pl.ds(base + j * BSZ, BSZ)], dst_v, sem).wait()
        pltpu.sync_copy(ones_v, acc.at[dst_v], add=True)

    plsc.subcore_barrier()

    @pl.when(cid == 0)
    def _():
        pltpu.sync_copy(acc.at[pl.ds(sid * RPT, RPT)],
                        d0_h.at[pl.ds(sid * RPT, RPT)])
        pltpu.sync_copy(zero_h.at[pl.ds(sid * TPT, TPT)],
                        d0_h.at[pl.ds(ACCR + sid * TPT, TPT)])

    @pl.when(cid == 1)
    def _():
        pltpu.sync_copy(acc.at[pl.ds(sid * RPT, RPT)],
                        d1_h.at[pl.ds(sid * RPT, RPT)])
        pltpu.sync_copy(zero_h.at[pl.ds(sid * TPT, TPT)],
                        d1_h.at[pl.ds(ACCR + sid * TPT, TPT)])


@functools.cache
def _deg_kernel():
    return functools.partial(
        pl.kernel,
        out_type=[jax.ShapeDtypeStruct((NPAD,), jnp.float32),
                  jax.ShapeDtypeStruct((NPAD,), jnp.float32)],
        mesh=_sc_mesh(),
        scratch_types=[
            pltpu.VMEM((BSZ,), jnp.int32),
            pltpu.VMEM((BSZ,), jnp.float32),
            pltpu.VMEM_SHARED((ACCR,), jnp.float32),
            pltpu.SemaphoreType.DMA,
        ],
        compiler_params=pltpu.CompilerParams(use_tc_tiling_on_sc=False),
    )(_deg_body)


def _deg(dstp, z1):
    return _deg_kernel()(dstp, z1)


def _make_agg_kernel(nc, fc):
    def body(*refs):
        g = refs[:nc]
        src_h, dst_h, zero_h = refs[nc:nc + 3]
        outs = refs[nc + 3:nc + 3 + nc]
        src_bulk, dst_v, msgs, acc, semg, semd = refs[nc + 3 + nc:]
        cid = lax.axis_index("c")
        sid = lax.axis_index("s")
        pltpu.sync_copy(src_h.at[pl.ds(sid * NBLK, NBLK)], src_bulk)
        for c in range(nc):
            @pl.when(cid == c % NC)
            def _(c=c):
                pltpu.sync_copy(zero_h.at[pl.ds(sid * RPT, RPT)],
                                acc.at[pl.ds(sid * RPT, RPT)])
                plsc.subcore_barrier()

                @pl.loop(0, NBLK)
                def _acc(j):
                    off = sid * EPT + j * BSZ
                    dcp = pltpu.async_copy(dst_h.at[pl.ds(off, BSZ)], dst_v,
                                           semd)
                    gcp = pltpu.async_copy(g[c].at[src_bulk.at[j]], msgs, semg)
                    gcp.wait()
                    dcp.wait()
                    pltpu.sync_copy(msgs, acc.at[dst_v], add=True)

                plsc.subcore_barrier()
                pltpu.sync_copy(acc.at[pl.ds(sid * RPT, RPT)],
                                outs[c].at[pl.ds(sid * RPT, RPT)])
                pltpu.sync_copy(zero_h.at[pl.ds(sid * TPT, TPT)],
                                outs[c].at[pl.ds(ACCR + sid * TPT, TPT)])

    return functools.partial(
        pl.kernel,
        out_type=[jax.ShapeDtypeStruct((NPAD, fc), jnp.float32)
                  for _ in range(nc)],
        mesh=_sc_mesh(),
        scratch_types=[
            pltpu.VMEM((NBLK, BSZ), jnp.int32),
            pltpu.VMEM((BSZ,), jnp.int32),
            pltpu.VMEM((BSZ, fc), jnp.float32),
            pltpu.VMEM_SHARED((ACCR, fc), jnp.float32),
            pltpu.SemaphoreType.DMA,
            pltpu.SemaphoreType.DMA,
        ],
        compiler_params=pltpu.CompilerParams(use_tc_tiling_on_sc=False),
    )(body)


_make_agg_kernel = functools.cache(_make_agg_kernel)

FC1, NC1 = 8, 4
FC2, NC2 = 16, 4
FC3, NC3 = 16, 8


def _agg1(*args):
    return _make_agg_kernel(NC1, FC1)(*args)


def _agg2(*args):
    return _make_agg_kernel(NC2, FC2)(*args)


def _agg3(*args):
    return _make_agg_kernel(NC3, FC3)(*args)


def _full(shape):
    return pl.BlockSpec(shape, lambda i: (0, 0))


def _rows(width):
    return pl.BlockSpec((RBLK, width), lambda i: (i, 0))


def _prep_body(x_ref, w_ref, d0_ref, d1_ref, *outs):
    dis = lax.rsqrt(1.0 + d0_ref[...] + d1_ref[...])
    g = jnp.dot(x_ref[...], w_ref[...],
                preferred_element_type=jnp.float32) * dis
    for i in range(NC1):
        outs[i][...] = g[:, i * FC1:(i + 1) * FC1]


def _prep(xp, w1p, d0, d1):
    return pl.pallas_call(
        _prep_body,
        grid=(NROWBLKS,),
        in_specs=[_rows(8), _full((8, 32)), _rows(1), _rows(1)],
        out_specs=[_rows(FC1)] * NC1,
        out_shape=[jax.ShapeDtypeStruct((NPAD, FC1), jnp.float32)
                   for _ in range(NC1)],
    )(xp, w1p, d0, d1)


def _make_layer(nc_in, fc_in, fin, fout, nc_out, fc_out):
    def body(*refs):
        chunks = refs[:nc_in]
        g_ref, d0_ref, d1_ref, b_ref, w_ref = refs[nc_in:nc_in + 5]
        outs = refs[nc_in + 5:]
        agg = jnp.concatenate([c[...] for c in chunks], axis=1)
        dis = lax.rsqrt(1.0 + d0_ref[...] + d1_ref[...])
        h = jnp.maximum(dis * (agg + g_ref[...]) + b_ref[...], 0.0)
        gn = jnp.dot(h, w_ref[...], preferred_element_type=jnp.float32) * dis
        for i in range(nc_out):
            outs[i][...] = gn[:, i * fc_out:(i + 1) * fc_out]

    def run(chunks, gl, d0, d1, b, w):
        return pl.pallas_call(
            body,
            grid=(NROWBLKS,),
            in_specs=([_rows(fc_in)] * nc_in
                      + [_rows(fin), _rows(1), _rows(1),
                         _full((1, fin)), _full((fin, fout))]),
            out_specs=[_rows(fc_out)] * nc_out,
            out_shape=[jax.ShapeDtypeStruct((NPAD, fc_out), jnp.float32)
                       for _ in range(nc_out)],
        )(*chunks, gl, d0, d1, b, w)

    return run


_layer1 = _make_layer(NC1, FC1, 32, 64, NC2, FC2)
_layer2 = _make_layer(NC2, FC2, 64, 128, NC3, FC3)


def _pool_body(*refs):
    chunks = refs[:NC3]
    g_ref, d0_ref, d1_ref, b_ref, batch_ref = refs[NC3:NC3 + 5]
    sums_ref, cnts_ref = refs[NC3 + 5:]

    @pl.when(pl.program_id(0) == 0)
    def _():
        sums_ref[...] = jnp.zeros_like(sums_ref)
        cnts_ref[...] = jnp.zeros_like(cnts_ref)

    agg = jnp.concatenate([c[...] for c in chunks], axis=1)
    dis = lax.rsqrt(1.0 + d0_ref[...] + d1_ref[...])
    h = jnp.maximum(dis * (agg + g_ref[...]) + b_ref[...], 0.0)
    p = (batch_ref[...] ==
         lax.broadcasted_iota(jnp.int32, (1, G), 1)).astype(jnp.float32)
    dn = (((0,), (0,)), ((), ()))
    sums_ref[...] += lax.dot_general(p, h, dn,
                                     preferred_element_type=jnp.float32)
    cnts_ref[...] += lax.dot_general(p, jnp.ones((RBLK, 1), jnp.float32), dn,
                                     preferred_element_type=jnp.float32)


def _pool(chunks, g3, d0, d1, b3, batchp):
    return pl.pallas_call(
        _pool_body,
        grid=(NROWBLKS,),
        in_specs=([_rows(FC3)] * NC3
                  + [_rows(128), _rows(1), _rows(1), _full((1, 128)),
                     _rows(1)]),
        out_specs=[_full((G, 128)), _full((G, 1))],
        out_shape=[jax.ShapeDtypeStruct((G, 128), jnp.float32),
                   jax.ShapeDtypeStruct((G, 1), jnp.float32)],
    )(*chunks, g3, d0, d1, b3, batchp)


def _mlp_body(s_ref, c_ref, w1_ref, b1_ref, w2_ref, b2_ref, o_ref):
    pooled = s_ref[...] / jnp.maximum(c_ref[...], 1.0)
    h = jnp.maximum(jnp.dot(pooled, w1_ref[...],
                            preferred_element_type=jnp.float32)
                    + b1_ref[...], 0.0)
    o_ref[...] = jnp.dot(h, w2_ref[...],
                         preferred_element_type=jnp.float32) + b2_ref[...]


def _mlp(sums, cnts, wf1, bf1, wf2, bf2):
    return pl.pallas_call(
        _mlp_body,
        grid=(1,),
        in_specs=[_full((G, 128)), _full((G, 1)), _full((128, 32)),
                  _full((1, 32)), _full((32, 4)), _full((1, 4))],
        out_specs=_full((G, 4)),
        out_shape=jax.ShapeDtypeStruct((G, 4), jnp.float32),
    )(sums, cnts, wf1, bf1, wf2, bf2)


def kernel(x, edge_index, batch, W1, b1, W2, b2, W3, b3, Wf1, bf1, Wf2, bf2):
    f32 = jnp.float32
    src = edge_index[0]
    dst = edge_index[1]
    pad = EP - E
    padv = jnp.full((pad,), N, jnp.int32)
    srcp = jnp.concatenate([src, padv]).reshape(EP // BSZ, BSZ)
    dstp = jnp.concatenate([dst, padv])
    xp = jnp.zeros((NPAD, 8), f32).at[:N, :3].set(x)
    w1p = jnp.zeros((8, 32), f32).at[:3].set(W1)
    batchp = jnp.concatenate(
        [batch, jnp.full((NPAD - N,), G, jnp.int32)]).reshape(NPAD, 1)
    z1 = jnp.zeros((NPAD,), f32)
    z8 = jnp.zeros((NPAD, FC1), f32)
    z16 = jnp.zeros((NPAD, FC2), f32)

    d0, d1 = _deg(dstp, z1)
    d0 = d0.reshape(NPAD, 1)
    d1 = d1.reshape(NPAD, 1)

    g1 = _prep(xp, w1p, d0, d1)
    a1 = _agg1(*g1, srcp, dstp, z8)
    g2 = _layer1(a1, jnp.concatenate(g1, axis=1), d0, d1,
                 b1.reshape(1, 32), W2)
    a2 = _agg2(*g2, srcp, dstp, z16)
    g3 = _layer2(a2, jnp.concatenate(g2, axis=1), d0, d1,
                 b2.reshape(1, 64), W3)
    a3 = _agg3(*g3, srcp, dstp, z16)
    sums, cnts = _pool(a3, jnp.concatenate(g3, axis=1), d0, d1,
                       b3.reshape(1, 128), batchp)
    return _mlp(sums, cnts, Wf1, bf1.reshape(1, 32), Wf2, bf2.reshape(1, 4))


# speedup vs baseline: 10.7740x; 1.4404x over previous
import functools

import jax
import jax.numpy as jnp
from jax import lax
from jax.experimental import pallas as pl
from jax.experimental.pallas import tpu as pltpu
from jax.experimental.pallas import tpu_sc as plsc

N = 50000
G = 64
E = 800000

NC = 2
NS = 16
BSZ = 128
NBLK = 392
EPT = NBLK * BSZ
EP = NS * EPT
DEG_NBLK = EP // (NC * NS * BSZ)

NPAD = 51200
RBLK = 2048
NROWBLKS = NPAD // RBLK
ACCR = 50048
RPT = ACCR // NS
TAILR = NPAD - ACCR
TPT = TAILR // NS


@functools.cache
def _sc_mesh():
    return plsc.VectorSubcoreMesh(core_axis_name="c", subcore_axis_name="s",
                                  num_cores=NC, num_subcores=NS)


def _deg_body(dst_h, zero_h, d0_h, d1_h, dst_v, ones_v, acc, sem):
    cid = lax.axis_index("c")
    sid = lax.axis_index("s")
    for i in range(BSZ // 16):
        ones_v[pl.ds(i * 16, 16)] = jnp.ones((16,), jnp.float32)
    pltpu.sync_copy(zero_h.at[pl.ds(sid * RPT, RPT)],
                    acc.at[pl.ds(sid * RPT, RPT)])
    plsc.subcore_barrier()
    base = cid * (EP // NC) + sid * (DEG_NBLK * BSZ)

    @pl.loop(0, DEG_NBLK)
    def _acc(j):
        pltpu.async_copy(dst_h.at[pl.ds(base + j * BSZ, BSZ)], dst_v, sem).wait()
        pltpu.sync_copy(ones_v, acc.at[dst_v], add=True)

    plsc.subcore_barrier()

    @pl.when(cid == 0)
    def _():
        pltpu.sync_copy(acc.at[pl.ds(sid * RPT, RPT)],
                        d0_h.at[pl.ds(sid * RPT, RPT)])
        pltpu.sync_copy(zero_h.at[pl.ds(sid * TPT, TPT)],
                        d0_h.at[pl.ds(ACCR + sid * TPT, TPT)])

    @pl.when(cid == 1)
    def _():
        pltpu.sync_copy(acc.at[pl.ds(sid * RPT, RPT)],
                        d1_h.at[pl.ds(sid * RPT, RPT)])
        pltpu.sync_copy(zero_h.at[pl.ds(sid * TPT, TPT)],
                        d1_h.at[pl.ds(ACCR + sid * TPT, TPT)])


@functools.cache
def _deg_kernel():
    return functools.partial(
        pl.kernel,
        out_type=[jax.ShapeDtypeStruct((NPAD,), jnp.float32),
                  jax.ShapeDtypeStruct((NPAD,), jnp.float32)],
        mesh=_sc_mesh(),
        scratch_types=[
            pltpu.VMEM((BSZ,), jnp.int32),
            pltpu.VMEM((BSZ,), jnp.float32),
            pltpu.VMEM_SHARED((ACCR,), jnp.float32),
            pltpu.SemaphoreType.DMA,
        ],
        compiler_params=pltpu.CompilerParams(use_tc_tiling_on_sc=False),
    )(_deg_body)


def _deg(dstp, z1):
    return _deg_kernel()(dstp, z1)


def _make_agg_kernel(nc, fc):
    npair = NBLK // 2

    def body(*refs):
        g = refs[:nc]
        src_h, dst_h, zero_h = refs[nc:nc + 3]
        outs = refs[nc + 3:nc + 3 + nc]
        (src_bulk, dst_a, dst_b, msg_a, msg_b, acc,
         sga, sgb, sda, sdb, ssa, ssb) = refs[nc + 3 + nc:]
        cid = lax.axis_index("c")
        sid = lax.axis_index("s")
        pltpu.sync_copy(src_h.at[pl.ds(sid * NBLK, NBLK)], src_bulk)

        for c in range(nc):
            @pl.when(cid == c % NC)
            def _(c=c):
                pltpu.sync_copy(zero_h.at[pl.ds(sid * RPT, RPT)],
                                acc.at[pl.ds(sid * RPT, RPT)])
                plsc.subcore_barrier()

                def start_a(j):
                    pltpu.async_copy(dst_h.at[pl.ds(sid * EPT + j * BSZ, BSZ)],
                                     dst_a, sda)
                    pltpu.async_copy(g[c].at[src_bulk.at[j]], msg_a, sga)

                start_a(0)

                @pl.loop(0, npair)
                def _pair(jj):
                    j0 = 2 * jj
                    j1 = j0 + 1
                    pltpu.async_copy(dst_h.at[pl.ds(sid * EPT + j1 * BSZ,
                                                    BSZ)], dst_b, sdb)
                    gb = pltpu.async_copy(g[c].at[src_bulk.at[j1]], msg_b, sgb)
                    pltpu.make_async_copy(dst_h.at[pl.ds(sid * EPT, BSZ)],
                                          dst_a, sda).wait()
                    pltpu.make_async_copy(g[c].at[src_bulk.at[j0]], msg_a,
                                          sga).wait()
                    sa = pltpu.async_copy(msg_a, acc.at[dst_a], ssa, add=True)
                    gb.wait()
                    pltpu.make_async_copy(dst_h.at[pl.ds(sid * EPT, BSZ)],
                                          dst_b, sdb).wait()
                    sa.wait()
                    sb = pltpu.async_copy(msg_b, acc.at[dst_b], ssb, add=True)

                    @pl.when(jj < npair - 1)
                    def _():
                        start_a(j0 + 2)

                    sb.wait()

                plsc.subcore_barrier()
                pltpu.sync_copy(acc.at[pl.ds(sid * RPT, RPT)],
                                outs[c].at[pl.ds(sid * RPT, RPT)])
                pltpu.sync_copy(zero_h.at[pl.ds(sid * TPT, TPT)],
                                outs[c].at[pl.ds(ACCR + sid * TPT, TPT)])

    return functools.partial(
        pl.kernel,
        out_type=[jax.ShapeDtypeStruct((NPAD, fc), jnp.float32)
                  for _ in range(nc)],
        mesh=_sc_mesh(),
        scratch_types=[
            pltpu.VMEM((NBLK, BSZ), jnp.int32),
            pltpu.VMEM((BSZ,), jnp.int32),
            pltpu.VMEM((BSZ,), jnp.int32),
            pltpu.VMEM((BSZ, fc), jnp.float32),
            pltpu.VMEM((BSZ, fc), jnp.float32),
            pltpu.VMEM_SHARED((ACCR, fc), jnp.float32),
            pltpu.SemaphoreType.DMA,
            pltpu.SemaphoreType.DMA,
            pltpu.SemaphoreType.DMA,
            pltpu.SemaphoreType.DMA,
            pltpu.SemaphoreType.DMA,
            pltpu.SemaphoreType.DMA,
        ],
        compiler_params=pltpu.CompilerParams(use_tc_tiling_on_sc=False),
    )(body)


_make_agg_kernel = functools.cache(_make_agg_kernel)

FC1, NC1 = 8, 4
FC2, NC2 = 16, 4
FC3, NC3 = 16, 8


def _agg1(*args):
    return _make_agg_kernel(NC1, FC1)(*args)


def _agg2(*args):
    return _make_agg_kernel(NC2, FC2)(*args)


def _agg3(*args):
    return _make_agg_kernel(NC3, FC3)(*args)


def _full(shape):
    return pl.BlockSpec(shape, lambda i: (0, 0))


def _rows(width):
    return pl.BlockSpec((RBLK, width), lambda i: (i, 0))


def _prep_body(x_ref, w_ref, d0_ref, d1_ref, *outs):
    dis = lax.rsqrt(1.0 + d0_ref[...] + d1_ref[...])
    g = jnp.dot(x_ref[...], w_ref[...],
                preferred_element_type=jnp.float32) * dis
    for i in range(NC1):
        outs[i][...] = g[:, i * FC1:(i + 1) * FC1]


def _prep(xp, w1p, d0, d1):
    return pl.pallas_call(
        _prep_body,
        grid=(NROWBLKS,),
        in_specs=[_rows(8), _full((8, 32)), _rows(1), _rows(1)],
        out_specs=[_rows(FC1)] * NC1,
        out_shape=[jax.ShapeDtypeStruct((NPAD, FC1), jnp.float32)
                   for _ in range(NC1)],
    )(xp, w1p, d0, d1)


def _make_layer(nc_in, fc_in, fin, fout, nc_out, fc_out):
    def body(*refs):
        chunks = refs[:nc_in]
        gs = refs[nc_in:2 * nc_in]
        d0_ref, d1_ref, b_ref, w_ref = refs[2 * nc_in:2 * nc_in + 4]
        outs = refs[2 * nc_in + 4:]
        agg = jnp.concatenate([c[...] for c in chunks], axis=1)
        gl = jnp.concatenate([c[...] for c in gs], axis=1)
        dis = lax.rsqrt(1.0 + d0_ref[...] + d1_ref[...])
        h = jnp.maximum(dis * (agg + gl) + b_ref[...], 0.0)
        gn = jnp.dot(h, w_ref[...], preferred_element_type=jnp.float32) * dis
        for i in range(nc_out):
            outs[i][...] = gn[:, i * fc_out:(i + 1) * fc_out]

    def run(chunks, gl, d0, d1, b, w):
        return pl.pallas_call(
            body,
            grid=(NROWBLKS,),
            in_specs=([_rows(fc_in)] * (2 * nc_in)
                      + [_rows(1), _rows(1),
                         _full((1, fin)), _full((fin, fout))]),
            out_specs=[_rows(fc_out)] * nc_out,
            out_shape=[jax.ShapeDtypeStruct((NPAD, fc_out), jnp.float32)
                       for _ in range(nc_out)],
        )(*chunks, *gl, d0, d1, b, w)

    return run


_layer1 = _make_layer(NC1, FC1, 32, 64, NC2, FC2)
_layer2 = _make_layer(NC2, FC2, 64, 128, NC3, FC3)


def _pool_body(*refs):
    chunks = refs[:NC3]
    gs = refs[NC3:2 * NC3]
    d0_ref, d1_ref, b_ref, batch_ref = refs[2 * NC3:2 * NC3 + 4]
    sums_ref, cnts_ref = refs[2 * NC3 + 4:]

    @pl.when(pl.program_id(0) == 0)
    def _():
        sums_ref[...] = jnp.zeros_like(sums_ref)
        cnts_ref[...] = jnp.zeros_like(cnts_ref)

    agg = jnp.concatenate([c[...] for c in chunks], axis=1)
    gl = jnp.concatenate([c[...] for c in gs], axis=1)
    dis = lax.rsqrt(1.0 + d0_ref[...] + d1_ref[...])
    h = jnp.maximum(dis * (agg + gl) + b_ref[...], 0.0)
    p = (batch_ref[...] ==
         lax.broadcasted_iota(jnp.int32, (1, G), 1)).astype(jnp.float32)
    dn = (((0,), (0,)), ((), ()))
    sums_ref[...] += lax.dot_general(p, h, dn,
                                     preferred_element_type=jnp.float32)
    cnts_ref[...] += lax.dot_general(p, jnp.ones((RBLK, 1), jnp.float32), dn,
                                     preferred_element_type=jnp.float32)


def _pool(chunks, g3, d0, d1, b3, batchp):
    return pl.pallas_call(
        _pool_body,
        grid=(NROWBLKS,),
        in_specs=([_rows(FC3)] * (2 * NC3)
                  + [_rows(1), _rows(1), _full((1, 128)), _rows(1)]),
        out_specs=[_full((G, 128)), _full((G, 1))],
        out_shape=[jax.ShapeDtypeStruct((G, 128), jnp.float32),
                   jax.ShapeDtypeStruct((G, 1), jnp.float32)],
    )(*chunks, *g3, d0, d1, b3, batchp)


def _mlp_body(s_ref, c_ref, w1_ref, b1_ref, w2_ref, b2_ref, o_ref):
    pooled = s_ref[...] / jnp.maximum(c_ref[...], 1.0)
    h = jnp.maximum(jnp.dot(pooled, w1_ref[...],
                            preferred_element_type=jnp.float32)
                    + b1_ref[...], 0.0)
    o_ref[...] = jnp.dot(h, w2_ref[...],
                         preferred_element_type=jnp.float32) + b2_ref[...]


def _mlp(sums, cnts, wf1, bf1, wf2, bf2):
    return pl.pallas_call(
        _mlp_body,
        grid=(1,),
        in_specs=[_full((G, 128)), _full((G, 1)), _full((128, 32)),
                  _full((1, 32)), _full((32, 4)), _full((1, 4))],
        out_specs=_full((G, 4)),
        out_shape=jax.ShapeDtypeStruct((G, 4), jnp.float32),
    )(sums, cnts, wf1, bf1, wf2, bf2)


def kernel(x, edge_index, batch, W1, b1, W2, b2, W3, b3, Wf1, bf1, Wf2, bf2):
    f32 = jnp.float32
    src = edge_index[0]
    dst = edge_index[1]
    pad = EP - E
    padv = jnp.full((pad,), N, jnp.int32)
    srcp = jnp.concatenate([src, padv]).reshape(EP // BSZ, BSZ)
    dstp = jnp.concatenate([dst, padv])
    xp = jnp.zeros((NPAD, 8), f32).at[:N, :3].set(x)
    w1p = jnp.zeros((8, 32), f32).at[:3].set(W1)
    batchp = jnp.concatenate(
        [batch, jnp.full((NPAD - N,), G, jnp.int32)]).reshape(NPAD, 1)
    z1 = jnp.zeros((NPAD,), f32)
    z8 = jnp.zeros((NPAD, FC1), f32)
    z16 = jnp.zeros((NPAD, FC2), f32)

    d0, d1 = _deg(dstp, z1)
    d0 = d0.reshape(NPAD, 1)
    d1 = d1.reshape(NPAD, 1)

    g1 = _prep(xp, w1p, d0, d1)
    a1 = _agg1(*g1, srcp, dstp, z8)
    g2 = _layer1(a1, g1, d0, d1, b1.reshape(1, 32), W2)
    a2 = _agg2(*g2, srcp, dstp, z16)
    g3 = _layer2(a2, g2, d0, d1, b2.reshape(1, 64), W3)
    a3 = _agg3(*g3, srcp, dstp, z16)
    sums, cnts = _pool(a3, g3, d0, d1, b3.reshape(1, 128), batchp)
    return _mlp(sums, cnts, Wf1, bf1.reshape(1, 32), Wf2, bf2.reshape(1, 4))


# speedup vs baseline: 18.3735x; 1.7054x over previous
import functools

import jax
import jax.numpy as jnp
from jax import lax
from jax.experimental import pallas as pl
from jax.experimental.pallas import tpu as pltpu
from jax.experimental.pallas import tpu_sc as plsc

N = 50000
G = 64
E = 800000

NC = 2
NS = 16
BSZ = 128
NBLK = 392
EPT = NBLK * BSZ
EP = NS * EPT
DEG_NBLK = EP // (NC * NS * BSZ)

NSLOT = 4
NPAD = 51200
RBLK = 2048
NROWBLKS = NPAD // RBLK
ACCR = 50048
RPT = ACCR // NS
TAILR = NPAD - ACCR
TPT = TAILR // NS


@functools.cache
def _sc_mesh():
    return plsc.VectorSubcoreMesh(core_axis_name="c", subcore_axis_name="s",
                                  num_cores=NC, num_subcores=NS)


def _deg_body(dst_h, zero_h, d0_h, d1_h, dst_v, ones_v, acc, sem):
    cid = lax.axis_index("c")
    sid = lax.axis_index("s")
    for i in range(BSZ // 16):
        ones_v[pl.ds(i * 16, 16)] = jnp.ones((16,), jnp.float32)
    pltpu.sync_copy(zero_h.at[pl.ds(sid * RPT, RPT)],
                    acc.at[pl.ds(sid * RPT, RPT)])
    plsc.subcore_barrier()
    base = cid * (EP // NC) + sid * (DEG_NBLK * BSZ)

    @pl.loop(0, DEG_NBLK)
    def _acc(j):
        pltpu.async_copy(dst_h.at[pl.ds(base + j * BSZ, BSZ)], dst_v, sem).wait()
        pltpu.sync_copy(ones_v, acc.at[dst_v], add=True)

    plsc.subcore_barrier()

    @pl.when(cid == 0)
    def _():
        pltpu.sync_copy(acc.at[pl.ds(sid * RPT, RPT)],
                        d0_h.at[pl.ds(sid * RPT, RPT)])
        pltpu.sync_copy(zero_h.at[pl.ds(sid * TPT, TPT)],
                        d0_h.at[pl.ds(ACCR + sid * TPT, TPT)])

    @pl.when(cid == 1)
    def _():
        pltpu.sync_copy(acc.at[pl.ds(sid * RPT, RPT)],
                        d1_h.at[pl.ds(sid * RPT, RPT)])
        pltpu.sync_copy(zero_h.at[pl.ds(sid * TPT, TPT)],
                        d1_h.at[pl.ds(ACCR + sid * TPT, TPT)])


@functools.cache
def _deg_kernel():
    return functools.partial(
        pl.kernel,
        out_type=[jax.ShapeDtypeStruct((NPAD,), jnp.float32),
                  jax.ShapeDtypeStruct((NPAD,), jnp.float32)],
        mesh=_sc_mesh(),
        scratch_types=[
            pltpu.VMEM((BSZ,), jnp.int32),
            pltpu.VMEM((BSZ,), jnp.float32),
            pltpu.VMEM_SHARED((ACCR,), jnp.float32),
            pltpu.SemaphoreType.DMA,
        ],
        compiler_params=pltpu.CompilerParams(use_tc_tiling_on_sc=False),
    )(_deg_body)


def _deg(dstp, z1):
    return _deg_kernel()(dstp, z1)


def _make_agg_kernel(nc, fc):
    nquad = NBLK // NSLOT

    def body(*refs):
        g = refs[:nc]
        src_h, dst_h, zero_h = refs[nc:nc + 3]
        outs = refs[nc + 3:nc + 3 + nc]
        scr = refs[nc + 3 + nc:]
        src_v = scr[0:NSLOT]
        dst_v = scr[NSLOT:2 * NSLOT]
        msg = scr[2 * NSLOT:3 * NSLOT]
        acc = scr[3 * NSLOT]
        sems = scr[3 * NSLOT + 1:]
        s_src = sems[0:NSLOT]
        s_dst = sems[NSLOT:2 * NSLOT]
        s_g = sems[2 * NSLOT:3 * NSLOT]
        s_sc = sems[3 * NSLOT:4 * NSLOT]
        cid = lax.axis_index("c")
        sid = lax.axis_index("s")

        for c in range(nc):
            @pl.when(cid == c % NC)
            def _(c=c):
                pltpu.sync_copy(zero_h.at[pl.ds(sid * RPT, RPT)],
                                acc.at[pl.ds(sid * RPT, RPT)])
                plsc.subcore_barrier()

                def idx_load(s, j):
                    off = sid * EPT + j * BSZ
                    pltpu.async_copy(src_h.at[pl.ds(off, BSZ)], src_v[s],
                                     s_src[s])
                    pltpu.async_copy(dst_h.at[pl.ds(off, BSZ)], dst_v[s],
                                     s_dst[s])

                def idx_wait(s):
                    off = sid * EPT
                    pltpu.make_async_copy(src_h.at[pl.ds(off, BSZ)], src_v[s],
                                          s_src[s]).wait()

                for s in range(NSLOT):
                    idx_load(s, s)

                @pl.loop(0, nquad)
                def _quad(jj):
                    q0 = NSLOT * jj
                    gs = []
                    for s in range(NSLOT):
                        idx_wait(s)
                        gs.append(pltpu.async_copy(g[c].at[src_v[s]], msg[s],
                                                   s_g[s]))
                    scs = []
                    for s in range(NSLOT):
                        gs[s].wait()
                        pltpu.make_async_copy(dst_h.at[pl.ds(sid * EPT, BSZ)],
                                              dst_v[s], s_dst[s]).wait()
                        scs.append(pltpu.async_copy(msg[s], acc.at[dst_v[s]],
                                                    s_sc[s], add=True))
                    for s in range(NSLOT):
                        scs[s].wait()

                        @pl.when(jj < nquad - 1)
                        def _(s=s):
                            idx_load(s, q0 + NSLOT + s)

                plsc.subcore_barrier()
                pltpu.sync_copy(acc.at[pl.ds(sid * RPT, RPT)],
                                outs[c].at[pl.ds(sid * RPT, RPT)])
                pltpu.sync_copy(zero_h.at[pl.ds(sid * TPT, TPT)],
                                outs[c].at[pl.ds(ACCR + sid * TPT, TPT)])

    return functools.partial(
        pl.kernel,
        out_type=[jax.ShapeDtypeStruct((NPAD, fc), jnp.float32)
                  for _ in range(nc)],
        mesh=_sc_mesh(),
        scratch_types=(
            [pltpu.VMEM((BSZ,), jnp.int32)] * (2 * NSLOT)
            + [pltpu.VMEM((BSZ, fc), jnp.float32)] * NSLOT
            + [pltpu.VMEM_SHARED((ACCR, fc), jnp.float32)]
            + [pltpu.SemaphoreType.DMA] * (4 * NSLOT)
        ),
        compiler_params=pltpu.CompilerParams(use_tc_tiling_on_sc=False),
    )(body)


_make_agg_kernel = functools.cache(_make_agg_kernel)

FCA = 32
FC1, NC1 = 32, 1
FC2, NC2 = 32, 2
FC3, NC3 = 32, 4


def _agg_pair(t0, t1, srcp, dstp, z):
    return _make_agg_kernel(2, FCA)(t0, t1, srcp, dstp, z)


def _full(shape):
    return pl.BlockSpec(shape, lambda i: (0, 0))


def _rows(width):
    return pl.BlockSpec((RBLK, width), lambda i: (i, 0))


def _prep_body(x_ref, w_ref, d0_ref, d1_ref, *outs):
    dis = lax.rsqrt(1.0 + d0_ref[...] + d1_ref[...])
    g = jnp.dot(x_ref[...], w_ref[...],
                preferred_element_type=jnp.float32) * dis
    for i in range(NC1):
        outs[i][...] = g[:, i * FC1:(i + 1) * FC1]


def _prep(xp, w1p, d0, d1):
    return pl.pallas_call(
        _prep_body,
        grid=(NROWBLKS,),
        in_specs=[_rows(8), _full((8, 32)), _rows(1), _rows(1)],
        out_specs=[_rows(FC1)] * NC1,
        out_shape=[jax.ShapeDtypeStruct((NPAD, FC1), jnp.float32)
                   for _ in range(NC1)],
    )(xp, w1p, d0, d1)


def _make_layer(nc_in, fc_in, fin, fout, nc_out, fc_out):
    def body(*refs):
        chunks = refs[:nc_in]
        gs = refs[nc_in:2 * nc_in]
        d0_ref, d1_ref, b_ref, w_ref = refs[2 * nc_in:2 * nc_in + 4]
        outs = refs[2 * nc_in + 4:]
        agg = jnp.concatenate([c[...] for c in chunks], axis=1)
        gl = jnp.concatenate([c[...] for c in gs], axis=1)
        dis = lax.rsqrt(1.0 + d0_ref[...] + d1_ref[...])
        h = jnp.maximum(dis * (agg + gl) + b_ref[...], 0.0)
        gn = jnp.dot(h, w_ref[...], preferred_element_type=jnp.float32) * dis
        for i in range(nc_out):
            outs[i][...] = gn[:, i * fc_out:(i + 1) * fc_out]

    def run(chunks, gl, d0, d1, b, w):
        return pl.pallas_call(
            body,
            grid=(NROWBLKS,),
            in_specs=([_rows(fc_in)] * (2 * nc_in)
                      + [_rows(1), _rows(1),
                         _full((1, fin)), _full((fin, fout))]),
            out_specs=[_rows(fc_out)] * nc_out,
            out_shape=[jax.ShapeDtypeStruct((NPAD, fc_out), jnp.float32)
                       for _ in range(nc_out)],
        )(*chunks, *gl, d0, d1, b, w)

    return run


_layer1 = _make_layer(NC1, FC1, 32, 64, NC2, FC2)
_layer2 = _make_layer(NC2, FC2, 64, 128, NC3, FC3)


def _pool_body(*refs):
    chunks = refs[:NC3]
    gs = refs[NC3:2 * NC3]
    d0_ref, d1_ref, b_ref, batch_ref = refs[2 * NC3:2 * NC3 + 4]
    sums_ref, cnts_ref = refs[2 * NC3 + 4:]

    @pl.when(pl.program_id(0) == 0)
    def _():
        sums_ref[...] = jnp.zeros_like(sums_ref)
        cnts_ref[...] = jnp.zeros_like(cnts_ref)

    agg = jnp.concatenate([c[...] for c in chunks], axis=1)
    gl = jnp.concatenate([c[...] for c in gs], axis=1)
    dis = lax.rsqrt(1.0 + d0_ref[...] + d1_ref[...])
    h = jnp.maximum(dis * (agg + gl) + b_ref[...], 0.0)
    p = (batch_ref[...] ==
         lax.broadcasted_iota(jnp.int32, (1, G), 1)).astype(jnp.float32)
    dn = (((0,), (0,)), ((), ()))
    sums_ref[...] += lax.dot_general(p, h, dn,
                                     preferred_element_type=jnp.float32)
    cnts_ref[...] += lax.dot_general(p, jnp.ones((RBLK, 1), jnp.float32), dn,
                                     preferred_element_type=jnp.float32)


def _pool(chunks, g3, d0, d1, b3, batchp):
    return pl.pallas_call(
        _pool_body,
        grid=(NROWBLKS,),
        in_specs=([_rows(FC3)] * (2 * NC3)
                  + [_rows(1), _rows(1), _full((1, 128)), _rows(1)]),
        out_specs=[_full((G, 128)), _full((G, 1))],
        out_shape=[jax.ShapeDtypeStruct((G, 128), jnp.float32),
                   jax.ShapeDtypeStruct((G, 1), jnp.float32)],
    )(*chunks, *g3, d0, d1, b3, batchp)


def _mlp_body(s_ref, c_ref, w1_ref, b1_ref, w2_ref, b2_ref, o_ref):
    pooled = s_ref[...] / jnp.maximum(c_ref[...], 1.0)
    h = jnp.maximum(jnp.dot(pooled, w1_ref[...],
                            preferred_element_type=jnp.float32)
                    + b1_ref[...], 0.0)
    o_ref[...] = jnp.dot(h, w2_ref[...],
                         preferred_element_type=jnp.float32) + b2_ref[...]


def _mlp(sums, cnts, wf1, bf1, wf2, bf2):
    return pl.pallas_call(
        _mlp_body,
        grid=(1,),
        in_specs=[_full((G, 128)), _full((G, 1)), _full((128, 32)),
                  _full((1, 32)), _full((32, 4)), _full((1, 4))],
        out_specs=_full((G, 4)),
        out_shape=jax.ShapeDtypeStruct((G, 4), jnp.float32),
    )(sums, cnts, wf1, bf1, wf2, bf2)


def kernel(x, edge_index, batch, W1, b1, W2, b2, W3, b3, Wf1, bf1, Wf2, bf2):
    f32 = jnp.float32
    src = edge_index[0]
    dst = edge_index[1]
    pad = EP - E
    padv = jnp.full((pad,), N, jnp.int32)
    srcp = jnp.concatenate([src, padv])
    dstp = jnp.concatenate([dst, padv])
    xp = jnp.zeros((NPAD, 8), f32).at[:N, :3].set(x)
    w1p = jnp.zeros((8, 32), f32).at[:3].set(W1)
    batchp = jnp.concatenate(
        [batch, jnp.full((NPAD - N,), G, jnp.int32)]).reshape(NPAD, 1)
    z1 = jnp.zeros((NPAD,), f32)
    z32 = jnp.zeros((NPAD, FCA), f32)

    d0, d1 = _deg(dstp, z1)
    d0 = d0.reshape(NPAD, 1)
    d1 = d1.reshape(NPAD, 1)

    g1 = _prep(xp, w1p, d0, d1)
    a1 = _agg_pair(g1[0], z32, srcp, dstp, z32)[:1]
    g2 = _layer1(a1, g1, d0, d1, b1.reshape(1, 32), W2)
    a2 = _agg_pair(g2[0], g2[1], srcp, dstp, z32)
    g3 = _layer2(a2, g2, d0, d1, b2.reshape(1, 64), W3)
    a3a = _agg_pair(g3[0], g3[1], srcp, dstp, z32)
    z32b = lax.optimization_barrier((z32, a3a[0]))[0]
    a3b = _agg_pair(g3[2], g3[3], srcp, dstp, z32b)
    a3 = list(a3a) + list(a3b)
    sums, cnts = _pool(a3, g3, d0, d1, b3.reshape(1, 128), batchp)
    return _mlp(sums, cnts, Wf1, bf1.reshape(1, 32), Wf2, bf2.reshape(1, 4))
